# baseline (speedup 1.0000x reference)
import functools

import jax
import jax.numpy as jnp
from jax import lax
from jax.experimental import pallas as pl
from jax.experimental.pallas import tpu as pltpu
from jax.experimental.pallas import tpu_sc as plsc

BATCH = 16384
LATENT = 64
NC = 2
NS = 16
NW = NC * NS
BPW = BATCH // NW
CHUNK = 128
NCHUNK = BPW // CHUNK


def _mf_kernel(uidx_hbm, iidx_hbm, uemb_hbm, iemb_hbm, ubias_hbm,
               ibias_hbm, gb_hbm, out_hbm,
               idx_u, idx_i, u_rows, v_rows, ub, ib, gbv, out_v, sem):
    wid = lax.axis_index("s") * NC + lax.axis_index("c")

    pltpu.sync_copy(uidx_hbm.at[pl.ds(wid * NCHUNK, NCHUNK)], idx_u)
    pltpu.sync_copy(iidx_hbm.at[pl.ds(wid * NCHUNK, NCHUNK)], idx_i)
    pltpu.sync_copy(gb_hbm, gbv)

    copies = []
    for j in range(NCHUNK):
        copies.append(pltpu.async_copy(
            uemb_hbm.at[idx_u.at[j]], u_rows.at[pl.ds(j * CHUNK, CHUNK)], sem))
        copies.append(pltpu.async_copy(
            iemb_hbm.at[idx_i.at[j]], v_rows.at[pl.ds(j * CHUNK, CHUNK)], sem))
        copies.append(pltpu.async_copy(
            ubias_hbm.at[idx_u.at[j]], ub.at[pl.ds(j * CHUNK, CHUNK)], sem))
        copies.append(pltpu.async_copy(
            ibias_hbm.at[idx_i.at[j]], ib.at[pl.ds(j * CHUNK, CHUNK)], sem))
    for c in copies:
        c.wait()

    lanes = lax.iota(jnp.int32, 16)
    gb_vec = gbv[...]

    def group(g, carry):
        base = pl.multiple_of(g * 16, 16)
        res = jnp.zeros((16,), jnp.float32)
        for s in range(16):
            row = base + s
            p = jnp.zeros((16,), jnp.float32)
            for k in range(LATENT // 16):
                uk = u_rows[row, pl.ds(k * 16, 16)]
                vk = v_rows[row, pl.ds(k * 16, 16)]
                p = p + uk * vk
            dot = jnp.sum(p)
            res = jnp.where(lanes == s, dot, res)
        bu = ub[pl.ds(base, 16)]
        bi = ib[pl.ds(base, 16)]
        out_v[pl.ds(base, 16)] = res + bu + bi + gb_vec
        return carry

    lax.fori_loop(0, BPW // 16, group, jnp.int32(0))

    pltpu.sync_copy(out_v, out_hbm.at[pl.ds(wid * BPW, BPW)])


@functools.partial(
    pl.kernel,
    out_type=jax.ShapeDtypeStruct((BATCH,), jnp.float32),
    mesh=plsc.VectorSubcoreMesh(core_axis_name="c", subcore_axis_name="s"),
    compiler_params=pltpu.CompilerParams(
        needs_layout_passes=False, use_tc_tiling_on_sc=False),
    scratch_types=[
        pltpu.VMEM((NCHUNK, CHUNK), jnp.int32),
        pltpu.VMEM((NCHUNK, CHUNK), jnp.int32),
        pltpu.VMEM((BPW, LATENT), jnp.float32),
        pltpu.VMEM((BPW, LATENT), jnp.float32),
        pltpu.VMEM((BPW,), jnp.float32),
        pltpu.VMEM((BPW,), jnp.float32),
        pltpu.VMEM((16,), jnp.float32),
        pltpu.VMEM((BPW,), jnp.float32),
        pltpu.SemaphoreType.DMA,
    ],
)
def _mf_call(*refs):
    _mf_kernel(*refs)


def kernel(user_idx, item_idx, user_emb, item_emb, user_bias, item_bias,
           global_bias):
    uidx = user_idx.astype(jnp.int32).reshape(NW * NCHUNK, CHUNK)
    iidx = item_idx.astype(jnp.int32).reshape(NW * NCHUNK, CHUNK)
    ub1 = user_bias.reshape(-1)
    ib1 = item_bias.reshape(-1)
    gb16 = jnp.broadcast_to(global_bias.astype(jnp.float32), (16,))
    return _mf_call(uidx, iidx, user_emb, item_emb, ub1, ib1, gb16)
